# baseline (speedup 1.0000x reference)
import functools

import jax
import jax.numpy as jnp
from jax import lax
from jax.experimental import pallas as pl
from jax.experimental.pallas import tpu as pltpu
from jax.experimental.pallas import tpu_sc as plsc

USER_COUNT = 1000000
ITEM_COUNT = 1000000
DIM = 32
BATCH = 16384

NUM_CORES = 2
NUM_SUBCORES = 16
NUM_WORKERS = NUM_CORES * NUM_SUBCORES
BPW = BATCH // NUM_WORKERS


def _mf_body(u_hbm, p_hbm, n_hbm, user_hbm, item_hbm,
             hu_hbm, hi_hbm, hj_hbm,
             uidx, pidx, nidx, hu_v, hi_v, hj_v, sem):
    wid = lax.axis_index("s") * NUM_CORES + lax.axis_index("c")
    base = wid * BPW
    pltpu.sync_copy(u_hbm.at[pl.ds(base, BPW)], uidx)
    pltpu.sync_copy(p_hbm.at[pl.ds(base, BPW)], pidx)
    pltpu.sync_copy(n_hbm.at[pl.ds(base, BPW)], nidx)
    cu = pltpu.async_copy(user_hbm.at[uidx], hu_v, sem)
    ci = pltpu.async_copy(item_hbm.at[pidx], hi_v, sem)
    cj = pltpu.async_copy(item_hbm.at[nidx], hj_v, sem)
    cu.wait()
    ci.wait()
    cj.wait()
    pltpu.sync_copy(hu_v, hu_hbm.at[pl.ds(base, BPW)])
    pltpu.sync_copy(hi_v, hi_hbm.at[pl.ds(base, BPW)])
    pltpu.sync_copy(hj_v, hj_hbm.at[pl.ds(base, BPW)])


@jax.jit
def kernel(u, p, n, user_emb, item_emb):
    u = jnp.asarray(u, jnp.int32)
    p = jnp.asarray(p, jnp.int32)
    n = jnp.asarray(n, jnp.int32)
    mesh = plsc.VectorSubcoreMesh(
        core_axis_name="c", subcore_axis_name="s",
        num_cores=NUM_CORES, num_subcores=NUM_SUBCORES)
    out = jax.ShapeDtypeStruct((BATCH, DIM), jnp.float32)
    run = pl.kernel(
        _mf_body,
        out_type=(out, out, out),
        mesh=mesh,
        scratch_types=[
            pltpu.VMEM((BPW,), jnp.int32),
            pltpu.VMEM((BPW,), jnp.int32),
            pltpu.VMEM((BPW,), jnp.int32),
            pltpu.VMEM((BPW, DIM), jnp.float32),
            pltpu.VMEM((BPW, DIM), jnp.float32),
            pltpu.VMEM((BPW, DIM), jnp.float32),
            pltpu.SemaphoreType.DMA,
        ],
        compiler_params=pltpu.CompilerParams(use_tc_tiling_on_sc=False),
    )
    return run(u, p, n, user_emb, item_emb)


# speedup vs baseline: 2.0522x; 2.0522x over previous
import functools

import jax
import jax.numpy as jnp
from jax import lax
from jax.experimental import pallas as pl
from jax.experimental.pallas import tpu as pltpu
from jax.experimental.pallas import tpu_sc as plsc

USER_COUNT = 1000000
ITEM_COUNT = 1000000
DIM = 32
BATCH = 16384
PAD = 128

NUM_CORES = 2
NUM_SUBCORES = 16
NUM_WORKERS = NUM_CORES * NUM_SUBCORES
BPW = BATCH // NUM_WORKERS
L = 16
WAVE = 8
KC = 128
NWAVE = BPW // WAVE


def _g_body(u_hbm, p_hbm, n_hbm, ut_hbm, it_hbm,
            ou, oi, oj, idxv, stg, obuf, sem):
    wid = lax.axis_index("s") * NUM_CORES + lax.axis_index("c")
    base = wid * BPW
    lane_iota = lax.iota(jnp.int32, L)

    def lookup(idx_hbm, table, out):
        pltpu.sync_copy(idx_hbm.at[pl.ds(base, BPW)], idxv)

        def wave(w, carry):
            wb = w * WAVE
            grp = idxv[pl.ds((wb // L) * L, L)]
            half = (wb % L) // WAVE
            copies = []
            rs = []
            for j in range(WAVE):
                lane = half * WAVE + j
                r = jnp.sum(jnp.where(lane_iota == lane, grp, 0))
                rs.append(r)
                tc = pl.multiple_of((r >> 7) << 7, 128)
                copies.append(pltpu.async_copy(
                    table.at[:, pl.ds(tc, 128)], stg.at[j], sem))
            for j in range(WAVE):
                copies[j].wait()
                l = rs[j] & 127
                lvec = jnp.broadcast_to(l, (L,))
                k = (wb % KC) + j
                lo = plsc.load_gather(stg.at[j], [lane_iota, lvec])
                hi = plsc.load_gather(stg.at[j], [lane_iota + L, lvec])
                obuf[k, pl.ds(0, L)] = lo
                obuf[k, pl.ds(L, L)] = hi

            @pl.when(lax.rem(w, KC // WAVE) == KC // WAVE - 1)
            def _():
                cb = (w // (KC // WAVE)) * KC
                pltpu.sync_copy(obuf, out.at[pl.ds(base + cb, KC)])

            return carry

        lax.fori_loop(0, NWAVE, wave, 0)

    lookup(u_hbm, ut_hbm, ou)
    lookup(p_hbm, it_hbm, oi)
    lookup(n_hbm, it_hbm, oj)


@jax.jit
def kernel(u, p, n, user_emb, item_emb):
    u = jnp.asarray(u, jnp.int32)
    p = jnp.asarray(p, jnp.int32)
    n = jnp.asarray(n, jnp.int32)
    ut = user_emb.T
    it = item_emb.T
    mesh = plsc.VectorSubcoreMesh(
        core_axis_name="c", subcore_axis_name="s",
        num_cores=NUM_CORES, num_subcores=NUM_SUBCORES)
    out = jax.ShapeDtypeStruct((BATCH, PAD), jnp.float32)
    run = pl.kernel(
        _g_body,
        out_type=(out, out, out),
        mesh=mesh,
        scratch_types=[
            pltpu.VMEM((BPW,), jnp.int32),
            pltpu.VMEM((WAVE, DIM, 128), jnp.float32),
            pltpu.VMEM((KC, PAD), jnp.float32),
            pltpu.SemaphoreType.DMA,
        ],
        compiler_params=pltpu.CompilerParams(needs_layout_passes=False),
    )
    ou, oi, oj = run(u, p, n, ut, it)
    return (ou[:, :DIM], oi[:, :DIM], oj[:, :DIM])


# speedup vs baseline: 2.5146x; 1.2253x over previous
import functools

import jax
import jax.numpy as jnp
from jax import lax
from jax.experimental import pallas as pl
from jax.experimental.pallas import tpu as pltpu
from jax.experimental.pallas import tpu_sc as plsc

USER_COUNT = 1000000
ITEM_COUNT = 1000000
DIM = 32
BATCH = 16384
PAD = 128

NUM_CORES = 2
NUM_SUBCORES = 16
NUM_WORKERS = NUM_CORES * NUM_SUBCORES
BPW = BATCH // NUM_WORKERS
L = 16
WAVE = 4
KC = 128
NWAVE = BPW // WAVE


def _g_body(u_hbm, p_hbm, n_hbm, ut_hbm, it_hbm,
            ou, oi, oj,
            iu, ip, inn, su, sp, sn, bu, bp, bn, semu, semp, semn):
    wid = lax.axis_index("s") * NUM_CORES + lax.axis_index("c")
    base = wid * BPW
    lane_iota = lax.iota(jnp.int32, L)

    streams = (
        (iu, ut_hbm, su, bu, ou, semu),
        (ip, it_hbm, sp, bp, oi, semp),
        (inn, it_hbm, sn, bn, oj, semn),
    )

    pltpu.sync_copy(u_hbm.at[pl.ds(base, BPW)], iu)
    pltpu.sync_copy(p_hbm.at[pl.ds(base, BPW)], ip)
    pltpu.sync_copy(n_hbm.at[pl.ds(base, BPW)], inn)

    def row_scalar(idxv, k):
        gb = pl.multiple_of((k >> 4) << 4, L)
        grp = idxv[pl.ds(gb, L)]
        return jnp.sum(jnp.where(lane_iota == (k & (L - 1)), grp, 0))

    def enqueue(idxv, table, stg, sem, w):
        for j in range(WAVE):
            r = row_scalar(idxv, w * WAVE + j)
            tc = pl.multiple_of((r >> 7) << 7, 128)
            pltpu.async_copy(table.at[:, pl.ds(tc, 128)], stg.at[j], sem)

    for idxv, table, stg, _, _, sem in streams:
        enqueue(idxv, table, stg, sem, 0)

    def wave(w, carry):
        for idxv, table, stg, obuf, out, sem in streams:
            for j in range(WAVE):
                pltpu.make_async_copy(
                    table.at[:, pl.ds(0, 128)], stg.at[j], sem).wait()
            for j in range(WAVE):
                r = row_scalar(idxv, w * WAVE + j)
                lvec = jnp.broadcast_to(r & 127, (L,))
                k = (w * WAVE + j) & (KC - 1)
                lo = plsc.load_gather(stg.at[j], [lane_iota, lvec])
                hi = plsc.load_gather(stg.at[j], [lane_iota + L, lvec])
                obuf[k, pl.ds(0, L)] = lo
                obuf[k, pl.ds(L, L)] = hi

            @pl.when(w < NWAVE - 1)
            def _():
                enqueue(idxv, table, stg, sem, w + 1)

            @pl.when(lax.rem(w, KC // WAVE) == KC // WAVE - 1)
            def _():
                cb = (w // (KC // WAVE)) * KC
                pltpu.sync_copy(obuf, out.at[pl.ds(base + cb, KC)])
        return carry

    lax.fori_loop(0, NWAVE, wave, 0)


@jax.jit
def kernel(u, p, n, user_emb, item_emb):
    u = jnp.asarray(u, jnp.int32)
    p = jnp.asarray(p, jnp.int32)
    n = jnp.asarray(n, jnp.int32)
    ut = user_emb.T
    it = item_emb.T
    mesh = plsc.VectorSubcoreMesh(
        core_axis_name="c", subcore_axis_name="s",
        num_cores=NUM_CORES, num_subcores=NUM_SUBCORES)
    out = jax.ShapeDtypeStruct((BATCH, PAD), jnp.float32)
    idx_t = pltpu.VMEM((BPW,), jnp.int32)
    stg_t = pltpu.VMEM((WAVE, DIM, 128), jnp.float32)
    obuf_t = pltpu.VMEM((KC, PAD), jnp.float32)
    run = pl.kernel(
        _g_body,
        out_type=(out, out, out),
        mesh=mesh,
        scratch_types=[
            idx_t, idx_t, idx_t,
            stg_t, stg_t, stg_t,
            obuf_t, obuf_t, obuf_t,
            pltpu.SemaphoreType.DMA,
            pltpu.SemaphoreType.DMA,
            pltpu.SemaphoreType.DMA,
        ],
        compiler_params=pltpu.CompilerParams(needs_layout_passes=False),
    )
    ou, oi, oj = run(u, p, n, ut, it)
    return (ou[:, :DIM], oi[:, :DIM], oj[:, :DIM])


# speedup vs baseline: 2.6338x; 1.0474x over previous
import functools

import jax
import jax.numpy as jnp
from jax import lax
from jax.experimental import pallas as pl
from jax.experimental.pallas import tpu as pltpu
from jax.experimental.pallas import tpu_sc as plsc

USER_COUNT = 1000000
ITEM_COUNT = 1000000
DIM = 32
BATCH = 16384
PAD = 128

NUM_CORES = 2
NUM_SUBCORES = 16
NUM_WORKERS = NUM_CORES * NUM_SUBCORES
BPW = BATCH // NUM_WORKERS
L = 16
WAVE = 8
KC = 64
NWAVE = BPW // WAVE


def _g_body(u_hbm, p_hbm, n_hbm, ut_hbm, it_hbm,
            ou, oi, oj,
            iu, ip, inn, su, sp, sn, bu, bp, bn, semu, semp, semn):
    wid = lax.axis_index("s") * NUM_CORES + lax.axis_index("c")
    base = wid * BPW
    lane_iota = lax.iota(jnp.int32, L)

    streams = (
        (iu, ut_hbm, su, bu, ou, semu),
        (ip, it_hbm, sp, bp, oi, semp),
        (inn, it_hbm, sn, bn, oj, semn),
    )

    pltpu.sync_copy(u_hbm.at[pl.ds(base, BPW)], iu)
    pltpu.sync_copy(p_hbm.at[pl.ds(base, BPW)], ip)
    pltpu.sync_copy(n_hbm.at[pl.ds(base, BPW)], inn)

    def row_scalar(idxv, k):
        gb = pl.multiple_of((k >> 4) << 4, L)
        grp = idxv[pl.ds(gb, L)]
        return jnp.sum(jnp.where(lane_iota == (k & (L - 1)), grp, 0))

    def enqueue(idxv, table, stg, sem, w):
        for j in range(WAVE):
            r = row_scalar(idxv, w * WAVE + j)
            tc = pl.multiple_of((r >> 7) << 7, 128)
            pltpu.async_copy(table.at[:, pl.ds(tc, 128)], stg.at[j], sem)

    for idxv, table, stg, _, _, sem in streams:
        enqueue(idxv, table, stg, sem, 0)

    def wave(w, carry):
        for idxv, table, stg, obuf, out, sem in streams:
            for j in range(WAVE):
                pltpu.make_async_copy(
                    table.at[:, pl.ds(0, 128)], stg.at[j], sem).wait()
            for j in range(WAVE):
                r = row_scalar(idxv, w * WAVE + j)
                lvec = jnp.broadcast_to(r & 127, (L,))
                k = (w * WAVE + j) & (KC - 1)
                lo = plsc.load_gather(stg.at[j], [lane_iota, lvec])
                hi = plsc.load_gather(stg.at[j], [lane_iota + L, lvec])
                obuf[k, pl.ds(0, L)] = lo
                obuf[k, pl.ds(L, L)] = hi

            @pl.when(w < NWAVE - 1)
            def _():
                enqueue(idxv, table, stg, sem, w + 1)

            @pl.when(lax.rem(w, KC // WAVE) == KC // WAVE - 1)
            def _():
                cb = (w // (KC // WAVE)) * KC
                pltpu.sync_copy(obuf, out.at[pl.ds(base + cb, KC)])
        return carry

    lax.fori_loop(0, NWAVE, wave, 0)


@jax.jit
def kernel(u, p, n, user_emb, item_emb):
    u = jnp.asarray(u, jnp.int32)
    p = jnp.asarray(p, jnp.int32)
    n = jnp.asarray(n, jnp.int32)
    ut = user_emb.T
    it = item_emb.T
    mesh = plsc.VectorSubcoreMesh(
        core_axis_name="c", subcore_axis_name="s",
        num_cores=NUM_CORES, num_subcores=NUM_SUBCORES)
    out = jax.ShapeDtypeStruct((BATCH, PAD), jnp.float32)
    idx_t = pltpu.VMEM((BPW,), jnp.int32)
    stg_t = pltpu.VMEM((WAVE, DIM, 128), jnp.float32)
    obuf_t = pltpu.VMEM((KC, PAD), jnp.float32)
    run = pl.kernel(
        _g_body,
        out_type=(out, out, out),
        mesh=mesh,
        scratch_types=[
            idx_t, idx_t, idx_t,
            stg_t, stg_t, stg_t,
            obuf_t, obuf_t, obuf_t,
            pltpu.SemaphoreType.DMA,
            pltpu.SemaphoreType.DMA,
            pltpu.SemaphoreType.DMA,
        ],
        compiler_params=pltpu.CompilerParams(needs_layout_passes=False),
    )
    ou, oi, oj = run(u, p, n, ut, it)
    return (ou[:, :DIM], oi[:, :DIM], oj[:, :DIM])
